# Initial kernel scaffold; baseline (speedup 1.0000x reference)
import jax
import jax.numpy as jnp
from jax.experimental import pallas as pl


def kernel(lig_x, lig_edge_index, pro_x, pro_edge_index, inter_edge_index, inter_edge_attr, lig_W, lig_b, pro_W, pro_b, W_e, b_e, W_fc1, b_fc1, W_fc2, b_fc2):
    raise NotImplementedError("write your pallas kernel here")


# speedup vs baseline: 4.0868x; 4.0868x over previous
import functools

import jax
import jax.numpy as jnp
from jax import lax
from jax.experimental import pallas as pl
from jax.experimental.pallas import tpu as pltpu
from jax.experimental.pallas import tpu_sc as plsc

N_NODES = 10000
N_ALL = 20000
E_EDGES = 320000
D = 128
EDIM = 16
FC_HID = 256
NC, NS = 2, 16
NW = NC * NS
EPW = E_EDGES // NW
CH = 80
NCHUNK = EPW // CH
RPT = 624
REM = N_NODES - NS * RPT


def _sc_mesh():
    return plsc.VectorSubcoreMesh(core_axis_name="c", subcore_axis_name="s")


def _sc_segsum_body(h_hbm, src_hbm, dst_hbm, zero_hbm, out_hbm,
                    idx_s, idx_d, rows, acc, gsem):
    c = lax.axis_index("c")
    s = lax.axis_index("s")
    pltpu.sync_copy(zero_hbm, acc.at[pl.ds(s * RPT, RPT)])

    @pl.when(s == NS - 1)
    def _():
        pltpu.sync_copy(zero_hbm.at[pl.ds(0, REM)],
                        acc.at[pl.ds(NS * RPT, REM)])

    pltpu.sync_copy(src_hbm.at[c, s], idx_s)
    pltpu.sync_copy(dst_hbm.at[c, s], idx_d)
    plsc.subcore_barrier()

    def body(j, carry):
        pltpu.async_copy(h_hbm.at[idx_s.at[j]], rows, gsem).wait()
        pltpu.sync_copy(rows, acc.at[idx_d.at[j]], add=True)
        return carry

    lax.fori_loop(0, NCHUNK, body, 0)
    plsc.subcore_barrier()
    pltpu.sync_copy(acc.at[pl.ds(s * RPT, RPT)],
                    out_hbm.at[c, pl.ds(s * RPT, RPT)])

    @pl.when(s == NS - 1)
    def _():
        pltpu.sync_copy(acc.at[pl.ds(NS * RPT, REM)],
                        out_hbm.at[c, pl.ds(NS * RPT, REM)])


def _sc_segsum(h, src4, dst4, zero):
    k = pl.kernel(
        _sc_segsum_body,
        out_type=jax.ShapeDtypeStruct((NC, N_NODES, D), jnp.float32),
        mesh=_sc_mesh(),
        scratch_types=[
            pltpu.VMEM((NCHUNK, CH), jnp.int32),
            pltpu.VMEM((NCHUNK, CH), jnp.int32),
            pltpu.VMEM((CH, D), jnp.float32),
            pltpu.VMEM_SHARED((N_NODES, D), jnp.float32),
            pltpu.SemaphoreType.DMA,
        ],
    )
    return k(h, src4, dst4, zero)


def _sc_gather2_body(ps_hbm, pd_hbm, src_hbm, dst_hbm, gs_hbm, gd_hbm,
                     idx_s, idx_d, rows_a, rows_b, gsem):
    c = lax.axis_index("c")
    s = lax.axis_index("s")
    pltpu.sync_copy(src_hbm.at[c, s], idx_s)
    pltpu.sync_copy(dst_hbm.at[c, s], idx_d)
    base = (c * NS + s) * EPW

    def body_s(j, carry):
        off = base + j * CH
        pltpu.async_copy(ps_hbm.at[idx_s.at[j]], rows_a, gsem).wait()
        pltpu.sync_copy(rows_a, gs_hbm.at[pl.ds(off, CH)])
        return carry

    def body_d(j, carry):
        off = base + j * CH
        pltpu.async_copy(pd_hbm.at[idx_d.at[j]], rows_b, gsem).wait()
        pltpu.sync_copy(rows_b, gd_hbm.at[pl.ds(off, CH)])
        return carry

    lax.fori_loop(0, NCHUNK, body_s, 0)
    lax.fori_loop(0, NCHUNK, body_d, 0)


def _sc_gather2(ps, pd, src4, dst4):
    k = pl.kernel(
        _sc_gather2_body,
        out_type=(
            jax.ShapeDtypeStruct((E_EDGES, D), jnp.float32),
            jax.ShapeDtypeStruct((E_EDGES, D), jnp.float32),
        ),
        mesh=_sc_mesh(),
        scratch_types=[
            pltpu.VMEM((NCHUNK, CH), jnp.int32),
            pltpu.VMEM((NCHUNK, CH), jnp.int32),
            pltpu.VMEM((CH, D), jnp.float32),
            pltpu.VMEM((CH, D), jnp.float32),
            pltpu.SemaphoreType.DMA,
        ],
    )
    return k(ps, pd, src4, dst4)


def _tc_layer_body(p_ref, w_ref, b_ref, o_ref):
    x = p_ref[0] + p_ref[1]
    y = jnp.dot(x, w_ref[...], preferred_element_type=jnp.float32,
                precision=lax.Precision.HIGHEST)
    o_ref[...] = jnp.maximum(y + b_ref[...], 0.0)


def _tc_layer(parts, W, b):
    BN = 1000
    return pl.pallas_call(
        _tc_layer_body,
        grid=(N_NODES // BN,),
        in_specs=[
            pl.BlockSpec((NC, BN, D), lambda i: (0, i, 0)),
            pl.BlockSpec((D, D), lambda i: (0, 0)),
            pl.BlockSpec((1, D), lambda i: (0, 0)),
        ],
        out_specs=pl.BlockSpec((BN, D), lambda i: (i, 0)),
        out_shape=jax.ShapeDtypeStruct((N_NODES, D), jnp.float32),
    )(parts, W, b.reshape(1, D))


def _tc_proj_body(h_ref, ws_ref, wd_ref, os_ref, od_ref):
    x = h_ref[...]
    os_ref[...] = jnp.dot(x, ws_ref[...], preferred_element_type=jnp.float32,
                precision=lax.Precision.HIGHEST)
    od_ref[...] = jnp.dot(x, wd_ref[...], preferred_element_type=jnp.float32,
                precision=lax.Precision.HIGHEST)


def _tc_proj(h_all, Ws, Wd):
    BN = 2000
    return pl.pallas_call(
        _tc_proj_body,
        grid=(N_ALL // BN,),
        in_specs=[
            pl.BlockSpec((BN, D), lambda i: (i, 0)),
            pl.BlockSpec((D, D), lambda i: (0, 0)),
            pl.BlockSpec((D, D), lambda i: (0, 0)),
        ],
        out_specs=(
            pl.BlockSpec((BN, D), lambda i: (i, 0)),
            pl.BlockSpec((BN, D), lambda i: (i, 0)),
        ),
        out_shape=(
            jax.ShapeDtypeStruct((N_ALL, D), jnp.float32),
            jax.ShapeDtypeStruct((N_ALL, D), jnp.float32),
        ),
    )(h_all, Ws, Wd)


_BE = 2560
_NEB = E_EDGES // _BE


def _tc_reduce_body(gs_ref, gd_ref, attr_ref, w3_ref, be_ref,
                    wf1_ref, bf1_ref, wf2_ref, bf2_ref,
                    ge_ref, aff_ref, sum_acc, max_acc):
    i = pl.program_id(0)
    e = (gs_ref[...] + gd_ref[...]
         + jnp.dot(attr_ref[...], w3_ref[...], preferred_element_type=jnp.float32,
                precision=lax.Precision.HIGHEST)
         + be_ref[...])
    y = jnp.where(e >= 0, e, 0.01 * e)
    bsum = jnp.sum(y, axis=0, keepdims=True)
    bmax = jnp.max(y, axis=0, keepdims=True)

    @pl.when(i == 0)
    def _():
        sum_acc[...] = bsum
        max_acc[...] = bmax

    @pl.when(i > 0)
    def _():
        sum_acc[...] += bsum
        max_acc[...] = jnp.maximum(max_acc[...], bmax)

    @pl.when(i == _NEB - 1)
    def _():
        ge = jnp.concatenate(
            [sum_acc[...] * (1.0 / E_EDGES), max_acc[...]], axis=1)
        h1 = jnp.maximum(
            jnp.dot(ge, wf1_ref[...], preferred_element_type=jnp.float32,
                precision=lax.Precision.HIGHEST)
            + bf1_ref[...], 0.0)
        aff = jnp.sum(h1 * wf2_ref[...], axis=1, keepdims=True) + bf2_ref[...]
        ge_ref[...] = ge
        aff_ref[...] = aff


def _tc_reduce(gs, gd, attr, W3, be, Wf1, bf1, Wf2, bf2):
    return pl.pallas_call(
        _tc_reduce_body,
        grid=(_NEB,),
        in_specs=[
            pl.BlockSpec((_BE, D), lambda i: (i, 0)),
            pl.BlockSpec((_BE, D), lambda i: (i, 0)),
            pl.BlockSpec((_BE, EDIM), lambda i: (i, 0)),
            pl.BlockSpec((EDIM, D), lambda i: (0, 0)),
            pl.BlockSpec((1, D), lambda i: (0, 0)),
            pl.BlockSpec((2 * D, FC_HID), lambda i: (0, 0)),
            pl.BlockSpec((1, FC_HID), lambda i: (0, 0)),
            pl.BlockSpec((1, FC_HID), lambda i: (0, 0)),
            pl.BlockSpec((1, 1), lambda i: (0, 0)),
        ],
        out_specs=(
            pl.BlockSpec((1, 2 * D), lambda i: (0, 0)),
            pl.BlockSpec((1, 1), lambda i: (0, 0)),
        ),
        out_shape=(
            jax.ShapeDtypeStruct((1, 2 * D), jnp.float32),
            jax.ShapeDtypeStruct((1, 1), jnp.float32),
        ),
        scratch_shapes=[
            pltpu.VMEM((1, D), jnp.float32),
            pltpu.VMEM((1, D), jnp.float32),
        ],
    )(gs, gd, attr, W3, be.reshape(1, D), Wf1, bf1.reshape(1, FC_HID),
      Wf2.reshape(1, FC_HID), bf2.reshape(1, 1))


def kernel(lig_x, lig_edge_index, pro_x, pro_edge_index, inter_edge_index,
           inter_edge_attr, lig_W, lig_b, pro_W, pro_b, W_e, b_e,
           W_fc1, b_fc1, W_fc2, b_fc2):
    zero = jnp.zeros((RPT, D), jnp.float32)

    def prep(ei):
        e = ei.astype(jnp.int32)
        return (e[0].reshape(NC, NS, NCHUNK, CH),
                e[1].reshape(NC, NS, NCHUNK, CH))

    lsrc, ldst = prep(lig_edge_index)
    qsrc, qdst = prep(pro_edge_index)
    isrc, idst = prep(inter_edge_index)

    h_lig = lig_x
    h_pro = pro_x
    for l in range(3):
        part = _sc_segsum(h_lig, lsrc, ldst, zero)
        h_lig = _tc_layer(part, lig_W[l], lig_b[l])
        part = _sc_segsum(h_pro, qsrc, qdst, zero)
        h_pro = _tc_layer(part, pro_W[l], pro_b[l])

    h_all = jnp.concatenate([h_lig, h_pro], axis=0)
    ps, pd = _tc_proj(h_all, W_e[:D], W_e[D:2 * D])
    gs, gd = _sc_gather2(ps, pd, isrc, idst)
    ge, aff = _tc_reduce(gs, gd, inter_edge_attr, W_e[2 * D:], b_e,
                         W_fc1, b_fc1, W_fc2, b_fc2)
    rank = jnp.zeros((1,), jnp.float32)
    return (aff, ge, rank)
